# baseline (device time: 8993 ns/iter reference)
import jax
import jax.numpy as jnp
from jax import lax
from jax.experimental import pallas as pl
from jax.experimental.pallas import tpu as pltpu

SUBS = (1536, 512)
NSUB = len(SUBS)


def kernel(x, W, labels):
    T, D = x.shape
    _, V = W.shape
    HV = V // 2
    OFFS = (0, SUBS[0])
    labels_row = labels.reshape(1, T)
    x = pltpu.with_memory_space_constraint(x, pltpu.HBM)
    W = pltpu.with_memory_space_constraint(W, pltpu.HBM)
    labels_row = pltpu.with_memory_space_constraint(labels_row, pltpu.HBM)

    def body(x_hbm, w_hbm, l_hbm, out_ref, xv, wbuf, lv, comm_send,
             comm_recv, x_sem, l_sem, w_sems, send_sems, recv_sems):
        my_x = lax.axis_index("x")
        my_y = lax.axis_index("y")
        my_z = lax.axis_index("z")
        peers = [
            (1 - my_x, my_y, my_z),
            (my_x, 1 - my_y, my_z),
            (1 - my_x, 1 - my_y, my_z),
        ]
        barrier = pltpu.get_barrier_semaphore()
        for p in peers:
            pl.semaphore_signal(barrier, inc=1, device_id=p,
                                device_id_type=pl.DeviceIdType.MESH)

        x_copy = pltpu.make_async_copy(x_hbm, xv, x_sem)
        x_copy.start()
        l_copy = pltpu.make_async_copy(l_hbm, lv, l_sem)
        l_copy.start()
        w_copies = [
            pltpu.make_async_copy(
                w_hbm.at[:, pl.ds(my_y * HV + OFFS[j], SUBS[j])],
                wbuf.at[j, :, 0:SUBS[j]], w_sems.at[j])
            for j in range(NSUB)
        ]
        for c in w_copies:
            c.start()
        x_copy.wait()
        l_copy.wait()

        barrier_done = False
        rdmas = []
        stats = []
        for j in range(NSUB):
            w_copies[j].wait()
            logits_t = lax.dot_general(
                wbuf[j, :, 0:SUBS[j]], xv[:, :],
                (((0,), (1,)), ((), ())),
                preferred_element_type=jnp.float32)
            s = jnp.sum(jnp.exp(logits_t), axis=0, keepdims=True)
            off = my_x * V + my_y * HV + OFFS[j]
            row = lax.broadcasted_iota(jnp.int32, (SUBS[j], T), 0) + off
            ll = jnp.sum(jnp.where(row == lv[:, :], logits_t, 0.0),
                         axis=0, keepdims=True)
            comm_send[j, 0:1, :] = s
            comm_send[j, 1:2, :] = ll
            stats.append((s, ll))
            if not barrier_done:
                pl.semaphore_wait(barrier, 3)
                barrier_done = True
            for i in range(3):
                r = pltpu.make_async_remote_copy(
                    src_ref=comm_send.at[j], dst_ref=comm_recv.at[i, j],
                    send_sem=send_sems.at[i, j],
                    recv_sem=recv_sems.at[i, j],
                    device_id=peers[i],
                    device_id_type=pl.DeviceIdType.MESH)
                r.start()
                rdmas.append(r)

        for r in rdmas:
            r.wait()

        s_all = stats[0][0]
        ll_all = stats[0][1]
        for s_i, ll_i in stats[1:]:
            s_all = s_all + s_i
            ll_all = ll_all + ll_i
        for i in range(3):
            for j in range(NSUB):
                s_all = s_all + comm_recv[i, j, 0:1, :]
                ll_all = ll_all + comm_recv[i, j, 1:2, :]
        nll = jnp.log(s_all) - ll_all
        out_ref[:] = nll[0, :]

    return pl.pallas_call(
        body,
        out_shape=jax.ShapeDtypeStruct((T,), jnp.float32),
        in_specs=[
            pl.BlockSpec(memory_space=pltpu.MemorySpace.HBM),
            pl.BlockSpec(memory_space=pltpu.MemorySpace.HBM),
            pl.BlockSpec(memory_space=pltpu.MemorySpace.HBM),
        ],
        out_specs=pl.BlockSpec(memory_space=pltpu.VMEM),
        scratch_shapes=[
            pltpu.VMEM((T, D), jnp.float32),
            pltpu.VMEM((NSUB, D, max(SUBS)), jnp.float32),
            pltpu.VMEM((1, T), jnp.int32),
            pltpu.VMEM((NSUB, 2, T), jnp.float32),
            pltpu.VMEM((3, NSUB, 2, T), jnp.float32),
            pltpu.SemaphoreType.DMA,
            pltpu.SemaphoreType.DMA,
            pltpu.SemaphoreType.DMA((NSUB,)),
            pltpu.SemaphoreType.DMA((3, NSUB)),
            pltpu.SemaphoreType.DMA((3, NSUB)),
        ],
        compiler_params=pltpu.CompilerParams(collective_id=0),
    )(x, W, labels_row)


# device time: 8736 ns/iter; 1.0294x vs baseline; 1.0294x over previous
import jax
import jax.numpy as jnp
from jax import lax
from jax.experimental import pallas as pl
from jax.experimental.pallas import tpu as pltpu

NSUB = 2


def kernel(x, W, labels):
    T, D = x.shape
    _, V = W.shape
    HV = V // 2
    SV = HV // NSUB
    labels_row = labels.reshape(1, T)
    x = pltpu.with_memory_space_constraint(x, pltpu.HBM)
    W = pltpu.with_memory_space_constraint(W, pltpu.HBM)
    labels_row = pltpu.with_memory_space_constraint(labels_row, pltpu.HBM)

    def body(x_hbm, w_hbm, l_hbm, out_ref, xv, wbuf, lv, comm_send,
             comm_recv, x_sem, l_sem, w_sems, send_sems, recv_sems):
        my_x = lax.axis_index("x")
        my_y = lax.axis_index("y")
        my_z = lax.axis_index("z")
        peers = [
            (1 - my_x, my_y, my_z),
            (my_x, 1 - my_y, my_z),
            (1 - my_x, 1 - my_y, my_z),
        ]
        barrier = pltpu.get_barrier_semaphore()
        for p in peers:
            pl.semaphore_signal(barrier, inc=1, device_id=p,
                                device_id_type=pl.DeviceIdType.MESH)

        x_copy = pltpu.make_async_copy(x_hbm, xv, x_sem)
        x_copy.start()
        l_copy = pltpu.make_async_copy(l_hbm, lv, l_sem)
        l_copy.start()
        w_copies = [
            pltpu.make_async_copy(
                w_hbm.at[:, pl.ds(my_y * HV + j * SV, SV)],
                wbuf.at[j], w_sems.at[j])
            for j in range(NSUB)
        ]
        for c in w_copies:
            c.start()
        x_copy.wait()
        l_copy.wait()

        barrier_done = False
        rdmas = []
        stats = []
        for j in range(NSUB):
            w_copies[j].wait()
            logits_t = lax.dot_general(
                wbuf[j].astype(jnp.bfloat16),
                xv[:, :].astype(jnp.bfloat16),
                (((0,), (1,)), ((), ())),
                preferred_element_type=jnp.float32)
            s = jnp.sum(jnp.exp(logits_t), axis=0, keepdims=True)
            off = my_x * V + my_y * HV + j * SV
            row = lax.broadcasted_iota(jnp.int32, (SV, T), 0) + off
            ll = jnp.sum(jnp.where(row == lv[:, :], logits_t, 0.0),
                         axis=0, keepdims=True)
            comm_send[j, 0:1, :] = s
            comm_send[j, 1:2, :] = ll
            stats.append((s, ll))
            if not barrier_done:
                pl.semaphore_wait(barrier, 3)
                barrier_done = True
            for i in range(3):
                r = pltpu.make_async_remote_copy(
                    src_ref=comm_send.at[j], dst_ref=comm_recv.at[i, j],
                    send_sem=send_sems.at[i, j],
                    recv_sem=recv_sems.at[i, j],
                    device_id=peers[i],
                    device_id_type=pl.DeviceIdType.MESH)
                r.start()
                rdmas.append(r)

        for r in rdmas:
            r.wait()

        s_all = stats[0][0]
        ll_all = stats[0][1]
        for s_i, ll_i in stats[1:]:
            s_all = s_all + s_i
            ll_all = ll_all + ll_i
        for i in range(3):
            for j in range(NSUB):
                s_all = s_all + comm_recv[i, j, 0:1, :]
                ll_all = ll_all + comm_recv[i, j, 1:2, :]
        nll = jnp.log(s_all) - ll_all
        out_ref[:] = nll[0, :]

    return pl.pallas_call(
        body,
        out_shape=jax.ShapeDtypeStruct((T,), jnp.float32),
        in_specs=[
            pl.BlockSpec(memory_space=pltpu.MemorySpace.HBM),
            pl.BlockSpec(memory_space=pltpu.MemorySpace.HBM),
            pl.BlockSpec(memory_space=pltpu.MemorySpace.HBM),
        ],
        out_specs=pl.BlockSpec(memory_space=pltpu.VMEM),
        scratch_shapes=[
            pltpu.VMEM((T, D), jnp.float32),
            pltpu.VMEM((NSUB, D, SV), jnp.float32),
            pltpu.VMEM((1, T), jnp.int32),
            pltpu.VMEM((NSUB, 2, T), jnp.float32),
            pltpu.VMEM((3, NSUB, 2, T), jnp.float32),
            pltpu.SemaphoreType.DMA,
            pltpu.SemaphoreType.DMA,
            pltpu.SemaphoreType.DMA((NSUB,)),
            pltpu.SemaphoreType.DMA((3, NSUB)),
            pltpu.SemaphoreType.DMA((3, NSUB)),
        ],
        compiler_params=pltpu.CompilerParams(collective_id=0),
    )(x, W, labels_row)


# device time: 8732 ns/iter; 1.0299x vs baseline; 1.0005x over previous
import jax
import jax.numpy as jnp
from jax import lax
from jax.experimental import pallas as pl
from jax.experimental.pallas import tpu as pltpu

NSUB = 2


def kernel(x, W, labels):
    T, D = x.shape
    _, V = W.shape
    HV = V // 2
    SV = HV // NSUB
    labels_row = labels.reshape(1, T)
    x = pltpu.with_memory_space_constraint(x, pltpu.HBM)
    W = pltpu.with_memory_space_constraint(W, pltpu.HBM)
    labels_row = pltpu.with_memory_space_constraint(labels_row, pltpu.HBM)

    def body(x_hbm, w_hbm, l_hbm, out_ref, xv, wbuf, lv, comm_send,
             comm_recv, x_sem, l_sem, w_sems, send_sems, recv_sems):
        my_x = lax.axis_index("x")
        my_y = lax.axis_index("y")
        my_z = lax.axis_index("z")
        peers = [
            (1 - my_x, my_y, my_z),
            (my_x, 1 - my_y, my_z),
            (1 - my_x, 1 - my_y, my_z),
        ]
        barrier = pltpu.get_barrier_semaphore()
        for p in peers:
            pl.semaphore_signal(barrier, inc=1, device_id=p,
                                device_id_type=pl.DeviceIdType.MESH)

        x_copy = pltpu.make_async_copy(x_hbm, xv, x_sem)
        x_copy.start()
        l_copy = pltpu.make_async_copy(l_hbm, lv, l_sem)
        l_copy.start()
        w_copies = [
            pltpu.make_async_copy(
                w_hbm.at[:, pl.ds(my_y * HV + j * SV, SV)],
                wbuf.at[j], w_sems.at[j])
            for j in range(NSUB)
        ]
        for c in w_copies:
            c.start()
        x_copy.wait()
        l_copy.wait()

        barrier_done = False
        rdmas = []
        stats = []
        for j in range(NSUB):
            w_copies[j].wait()
            logits_t = lax.dot_general(
                wbuf[j].astype(jnp.bfloat16),
                xv[:, :].astype(jnp.bfloat16),
                (((0,), (1,)), ((), ())),
                preferred_element_type=jnp.float32)
            s = jnp.sum(jnp.exp(logits_t), axis=0, keepdims=True)
            off = my_x * V + my_y * HV + j * SV
            row = lax.broadcasted_iota(jnp.int32, (SV, T), 0) + off
            ll = jnp.sum(jnp.where(row == lv[:, :], logits_t, 0.0),
                         axis=0, keepdims=True)
            comm_send[j, 0:1, :] = s
            comm_send[j, 1:2, :] = ll
            stats.append((s, ll))
            if not barrier_done:
                pl.semaphore_wait(barrier, 3)
                barrier_done = True
            for i in range(3):
                r = pltpu.make_async_remote_copy(
                    src_ref=comm_send.at[j], dst_ref=comm_recv.at[i, j],
                    send_sem=send_sems.at[i, j],
                    recv_sem=recv_sems.at[i, j],
                    device_id=peers[i],
                    device_id_type=pl.DeviceIdType.MESH)
                r.start()
                rdmas.append(r)

        for r in rdmas:
            r.wait()

        s_all = stats[0][0]
        ll_all = stats[0][1]
        for s_i, ll_i in stats[1:]:
            s_all = s_all + s_i
            ll_all = ll_all + ll_i
        for i in range(3):
            for j in range(NSUB):
                s_all = s_all + comm_recv[i, j, 0:1, :]
                ll_all = ll_all + comm_recv[i, j, 1:2, :]
        nll = jnp.log(s_all) - ll_all
        out_ref[:] = nll[0, :]

    return pl.pallas_call(
        body,
        out_shape=jax.ShapeDtypeStruct((T,), jnp.float32),
        in_specs=[
            pl.BlockSpec(memory_space=pltpu.MemorySpace.HBM),
            pl.BlockSpec(memory_space=pltpu.MemorySpace.HBM),
            pl.BlockSpec(memory_space=pltpu.MemorySpace.HBM),
        ],
        out_specs=pl.BlockSpec(memory_space=pltpu.VMEM),
        scratch_shapes=[
            pltpu.VMEM((T, D), jnp.float32),
            pltpu.VMEM((NSUB, D, SV), jnp.float32),
            pltpu.VMEM((1, T), jnp.int32),
            pltpu.VMEM((NSUB, 2, T), jnp.float32),
            pltpu.VMEM((3, NSUB, 2, T), jnp.float32),
            pltpu.SemaphoreType.DMA,
            pltpu.SemaphoreType.DMA,
            pltpu.SemaphoreType.DMA((NSUB,)),
            pltpu.SemaphoreType.DMA((3, NSUB)),
            pltpu.SemaphoreType.DMA((3, NSUB)),
        ],
        compiler_params=pltpu.CompilerParams(
            collective_id=0,
            disable_bounds_checks=True,
        ),
    )(x, W, labels_row)
